# baseline (speedup 1.0000x reference)
import functools

import jax
import jax.numpy as jnp
from jax import lax
from jax.experimental import pallas as pl
from jax.experimental.pallas import tpu as pltpu
from jax.experimental.pallas import tpu_sc as plsc

_N = 10000
_E = 320000
_NC = 2
_NS = 16
_CHUNK = 80
_EBLK = 10000
_NCHUNK = _EBLK // _CHUNK
_RPT = _N // _NS
_ZR = 25
_R = 2000


def _mm_first_body(x_ref, w_ref, o_ref):
    h = jnp.dot(x_ref[...], w_ref[...], preferred_element_type=jnp.float32)
    o_ref[0, :, :] = h[:, :128]
    o_ref[1, :, :] = h[:, 128:]


def _mm_first(x, w):
    return pl.pallas_call(
        _mm_first_body,
        grid=(_N // _R,),
        in_specs=[pl.BlockSpec((_R, 128), lambda i: (i, 0)),
                  pl.BlockSpec((128, 256), lambda i: (0, 0))],
        out_specs=pl.BlockSpec((2, _R, 128), lambda i: (0, i, 0)),
        out_shape=jax.ShapeDtypeStruct((2, _N, 128), jnp.float32),
    )(x, w)


def _mm_mid_body(a_ref, w_ref, o_ref):
    x = jnp.concatenate([a_ref[0, :, :], a_ref[1, :, :]], axis=1)
    h = jnp.dot(jnp.tanh(x), w_ref[...], preferred_element_type=jnp.float32)
    o_ref[0, :, :] = h[:, :128]
    o_ref[1, :, :] = h[:, 128:]


def _mm_mid(a, w):
    return pl.pallas_call(
        _mm_mid_body,
        grid=(_N // _R,),
        in_specs=[pl.BlockSpec((2, _R, 128), lambda i: (0, i, 0)),
                  pl.BlockSpec((256, 256), lambda i: (0, 0))],
        out_specs=pl.BlockSpec((2, _R, 128), lambda i: (0, i, 0)),
        out_shape=jax.ShapeDtypeStruct((2, _N, 128), jnp.float32),
    )(a, w)


def _mm_full_body(a_ref, w_ref, o_ref):
    x = jnp.concatenate([a_ref[0, :, :], a_ref[1, :, :]], axis=1)
    o_ref[...] = jnp.dot(jnp.tanh(x), w_ref[...],
                         preferred_element_type=jnp.float32)


def _mm_full(a, w):
    dout = w.shape[1]
    return pl.pallas_call(
        _mm_full_body,
        grid=(_N // _R,),
        in_specs=[pl.BlockSpec((2, _R, 128), lambda i: (0, i, 0)),
                  pl.BlockSpec(w.shape, lambda i: (0, 0))],
        out_specs=pl.BlockSpec((_R, dout), lambda i: (i, 0)),
        out_shape=jax.ShapeDtypeStruct((_N, dout), jnp.float32),
    )(a, w)


def _tanh_sum_body(a_ref, o_ref):
    o_ref[...] = jnp.tanh(a_ref[0, :, :] + a_ref[1, :, :])


def _final_tanh_sum(a):
    return pl.pallas_call(
        _tanh_sum_body,
        grid=(_N // _R,),
        in_specs=[pl.BlockSpec((2, _R, 128), lambda i: (0, i, 0))],
        out_specs=pl.BlockSpec((_R, 128), lambda i: (i, 0)),
        out_shape=jax.ShapeDtypeStruct((_N, 128), jnp.float32),
    )(a)


@functools.lru_cache(maxsize=None)
def _make_agg(edge_split):
    chunk = 80
    nchunk = 50 if not edge_split else 25
    eblk = nchunk * chunk
    ept = (_E // (_NC * _NS)) if edge_split else (_E // _NS)
    nblk = ept // eblk
    mesh = plsc.VectorSubcoreMesh(core_axis_name="c", subcore_axis_name="s")

    @functools.partial(
        pl.kernel, mesh=mesh,
        out_type=jax.ShapeDtypeStruct((_NC * _N, 128), jnp.float32),
        scratch_types=[
            pltpu.VMEM((eblk,), jnp.int32),
            pltpu.VMEM((eblk,), jnp.int32),
            pltpu.VMEM((chunk, 128), jnp.float32),
            pltpu.VMEM((chunk, 128), jnp.float32),
            pltpu.VMEM((chunk, 128), jnp.float32),
            pltpu.VMEM((chunk, 128), jnp.float32),
            pltpu.VMEM_SHARED((_N, 128), jnp.float32),
            pltpu.SemaphoreType.DMA,
            pltpu.SemaphoreType.DMA,
            pltpu.SemaphoreType.DMA,
            pltpu.SemaphoreType.DMA,
            pltpu.SemaphoreType.DMA,
            pltpu.SemaphoreType.DMA,
            pltpu.SemaphoreType.DMA,
            pltpu.SemaphoreType.DMA,
        ],
    )
    def agg(h0_hbm, h1_hbm, src_hbm, dst_hbm, out_hbm, sidx, didx, rows_0,
            rows_1, rows_2, rows_3, accum, sg0, sg1, sg2, sg3, ss0, ss1,
            ss2, ss3):
        bufs = (rows_0, rows_1, rows_2, rows_3)
        sem_g = (sg0, sg1, sg2, sg3)
        sem_s = (ss0, ss1, ss2, ss3)
        cid = lax.axis_index("c")
        sid = lax.axis_index("s")

        def zrow(r, _):
            def zcol(j, _):
                rows_0[r, pl.ds(j * 16, 16)] = jnp.zeros((16,), jnp.float32)
                return 0
            return lax.fori_loop(0, 128 // 16, zcol, 0)
        lax.fori_loop(0, chunk, zrow, 0)
        rbase = sid * _RPT
        nz = _RPT // chunk
        for z in range(nz):
            pltpu.sync_copy(rows_0, accum.at[pl.ds(rbase + z * chunk, chunk)])
        rem = _RPT - nz * chunk
        pltpu.sync_copy(rows_0.at[pl.ds(0, rem)],
                        accum.at[pl.ds(rbase + nz * chunk, rem)])
        plsc.subcore_barrier()

        off0 = cid * _N

        def gather(i, j):
            idx = sidx.at[pl.ds(i * chunk, chunk)]

            @pl.when(cid == 0)
            def _g0():
                pltpu.make_async_copy(h0_hbm.at[idx], bufs[j], sem_g[j]).start()

            @pl.when(cid == 1)
            def _g1():
                pltpu.make_async_copy(h1_hbm.at[idx], bufs[j], sem_g[j]).start()

        def wait_gather(i, j):
            idx = sidx.at[pl.ds(i * chunk, chunk)]
            pltpu.make_async_copy(h0_hbm.at[idx], bufs[j], sem_g[j]).wait()

        def scatter(i, j):
            idx = didx.at[pl.ds(i * chunk, chunk)]
            pltpu.async_copy(bufs[j], accum.at[idx], sem_s[j], add=True)

        def wait_scatter(i, j):
            idx = didx.at[pl.ds(i * chunk, chunk)]
            pltpu.make_async_copy(bufs[j], accum.at[idx], sem_s[j]).wait()

        def chunk_body(i, j, first):
            wait_gather(i, j)
            scatter(i, j)
            if not first:
                wait_scatter(i - 2, (j + 2) % 4)

        for blk in range(nblk):
            if edge_split:
                ebase = cid * (_E // _NC) + sid * ept + blk * eblk
            else:
                ebase = sid * ept + blk * eblk
            pltpu.sync_copy(src_hbm.at[pl.ds(ebase, eblk)], sidx)
            pltpu.sync_copy(dst_hbm.at[pl.ds(ebase, eblk)], didx)

            gather(0, 0)
            gather(1, 1)
            for i in (0, 1):
                chunk_body(i, i, True)
                gather(i + 2, i + 2)

            nfull = (nchunk - 2) // 4

            def quad(q, _):
                for j in range(4):
                    i = 2 + q * 4 + j
                    bj = (2 + j) % 4
                    chunk_body(i, bj, False)

                    @pl.when(i + 2 < nchunk)
                    def _nx():
                        gather(i + 2, (bj + 2) % 4)
                return 0
            lax.fori_loop(0, nfull, quad, 0)
            for j in range(nchunk - 2 - nfull * 4):
                i = 2 + nfull * 4 + j
                chunk_body(i, i % 4, False)
                if i + 2 < nchunk:
                    gather(i + 2, (i + 2) % 4)
            wait_scatter(nchunk - 2, (nchunk - 2) % 4)
            wait_scatter(nchunk - 1, (nchunk - 1) % 4)
        plsc.subcore_barrier()

        cbase = sid * 624
        pltpu.sync_copy(accum.at[pl.ds(cbase, 624)],
                        out_hbm.at[pl.ds(off0 + cbase, 624)])

        @pl.when(sid == _NS - 1)
        def _tail():
            pltpu.sync_copy(accum.at[pl.ds(9984, 16)],
                            out_hbm.at[pl.ds(off0 + 9984, 16)])

    return agg


def kernel(features, edge_index, W0, W1, W2):
    src = edge_index[0].astype(jnp.int32)
    dst = edge_index[1].astype(jnp.int32)
    fagg = _make_agg(False)
    eagg = _make_agg(True)
    hp = _mm_first(features, W0)
    a0 = fagg(hp[0], hp[1], src, dst).reshape(_NC, _N, 128)
    hp = _mm_mid(a0, W1)
    a1 = fagg(hp[0], hp[1], src, dst).reshape(_NC, _N, 128)
    h2 = _mm_full(a1, W2)
    a2 = eagg(h2, h2, src, dst).reshape(_NC, _N, 128)
    return _final_tanh_sum(a2)


# speedup vs baseline: 1.0456x; 1.0456x over previous
import functools

import jax
import jax.numpy as jnp
from jax import lax
from jax.experimental import pallas as pl
from jax.experimental.pallas import tpu as pltpu
from jax.experimental.pallas import tpu_sc as plsc

_N = 10000
_E = 320000
_NC = 2
_NS = 16
_RPT = _N // _NS
_R = 2000


def _mm_first_body(x_ref, w_ref, o_ref):
    h = jnp.dot(x_ref[...], w_ref[...], preferred_element_type=jnp.float32)
    o_ref[0, :, :] = h[:, :128]
    o_ref[1, :, :] = h[:, 128:]


def _mm_first(x, w):
    return pl.pallas_call(
        _mm_first_body,
        grid=(_N // _R,),
        in_specs=[pl.BlockSpec((_R, 128), lambda i: (i, 0)),
                  pl.BlockSpec((128, 256), lambda i: (0, 0))],
        out_specs=pl.BlockSpec((2, _R, 128), lambda i: (0, i, 0)),
        out_shape=jax.ShapeDtypeStruct((2, _N, 128), jnp.float32),
    )(x, w)


def _mm_mid_body(a_ref, w_ref, o_ref):
    x = jnp.concatenate([a_ref[0, :, :], a_ref[1, :, :]], axis=1)
    h = jnp.dot(jnp.tanh(x), w_ref[...], preferred_element_type=jnp.float32)
    o_ref[0, :, :] = h[:, :128]
    o_ref[1, :, :] = h[:, 128:]


def _mm_mid(a, w):
    return pl.pallas_call(
        _mm_mid_body,
        grid=(_N // _R,),
        in_specs=[pl.BlockSpec((2, _R, 128), lambda i: (0, i, 0)),
                  pl.BlockSpec((256, 256), lambda i: (0, 0))],
        out_specs=pl.BlockSpec((2, _R, 128), lambda i: (0, i, 0)),
        out_shape=jax.ShapeDtypeStruct((2, _N, 128), jnp.float32),
    )(a, w)


def _mm_full_body(a_ref, w_ref, o_ref):
    x = jnp.concatenate([a_ref[0, :, :], a_ref[1, :, :]], axis=1)
    o_ref[...] = jnp.dot(jnp.tanh(x), w_ref[...],
                         preferred_element_type=jnp.float32)


def _mm_full(a, w):
    dout = w.shape[1]
    return pl.pallas_call(
        _mm_full_body,
        grid=(_N // _R,),
        in_specs=[pl.BlockSpec((2, _R, 128), lambda i: (0, i, 0)),
                  pl.BlockSpec(w.shape, lambda i: (0, 0))],
        out_specs=pl.BlockSpec((_R, dout), lambda i: (i, 0)),
        out_shape=jax.ShapeDtypeStruct((_N, dout), jnp.float32),
    )(a, w)


def _tanh_sum_body(a_ref, o_ref):
    o_ref[...] = jnp.tanh(a_ref[0, :, :] + a_ref[1, :, :])


def _final_tanh_sum(a):
    return pl.pallas_call(
        _tanh_sum_body,
        grid=(_N // _R,),
        in_specs=[pl.BlockSpec((2, _R, 128), lambda i: (0, i, 0))],
        out_specs=pl.BlockSpec((_R, 128), lambda i: (i, 0)),
        out_shape=jax.ShapeDtypeStruct((_N, 128), jnp.float32),
    )(a)


@functools.lru_cache(maxsize=None)
def _make_agg(edge_split):
    chunk = 80 if edge_split else 160
    nchunk = 25
    eblk = nchunk * chunk
    ept = (_E // (_NC * _NS)) if edge_split else (_E // _NS)
    nblk = ept // eblk
    mesh = plsc.VectorSubcoreMesh(core_axis_name="c", subcore_axis_name="s")

    @functools.partial(
        pl.kernel, mesh=mesh,
        out_type=jax.ShapeDtypeStruct((_NC * _N, 128), jnp.float32),
        scratch_types=[
            pltpu.VMEM((eblk,), jnp.int32),
            pltpu.VMEM((eblk,), jnp.int32),
            pltpu.VMEM((chunk, 128), jnp.float32),
            pltpu.VMEM((chunk, 128), jnp.float32),
            pltpu.VMEM_SHARED((_N, 128), jnp.float32),
            pltpu.SemaphoreType.DMA,
            pltpu.SemaphoreType.DMA,
        ],
    )
    def agg(h0_hbm, h1_hbm, src_hbm, dst_hbm, out_hbm, sidx, didx, rows_a,
            rows_b, accum, sem_a, sem_b):
        cid = lax.axis_index("c")
        sid = lax.axis_index("s")

        def zrow(r, _):
            def zcol(j, _):
                rows_a[r, pl.ds(j * 16, 16)] = jnp.zeros((16,), jnp.float32)
                return 0
            return lax.fori_loop(0, 128 // 16, zcol, 0)
        lax.fori_loop(0, chunk, zrow, 0)
        rbase = sid * _RPT
        nz = _RPT // chunk
        for z in range(nz):
            pltpu.sync_copy(rows_a, accum.at[pl.ds(rbase + z * chunk, chunk)])
        rem = _RPT - nz * chunk
        pltpu.sync_copy(rows_a.at[pl.ds(0, rem)],
                        accum.at[pl.ds(rbase + nz * chunk, rem)])
        plsc.subcore_barrier()

        off0 = cid * _N

        def gather(i, buf, sem):
            idx = sidx.at[pl.ds(i * chunk, chunk)]

            @pl.when(cid == 0)
            def _g0():
                pltpu.make_async_copy(h0_hbm.at[idx], buf, sem).start()

            @pl.when(cid == 1)
            def _g1():
                pltpu.make_async_copy(h1_hbm.at[idx], buf, sem).start()

        def wait_gather(i, buf, sem):
            idx = sidx.at[pl.ds(i * chunk, chunk)]
            pltpu.make_async_copy(h0_hbm.at[idx], buf, sem).wait()

        def scatter(i, buf):
            idx = didx.at[pl.ds(i * chunk, chunk)]
            pltpu.sync_copy(buf, accum.at[idx], add=True)

        for blk in range(nblk):
            if edge_split:
                ebase = cid * (_E // _NC) + sid * ept + blk * eblk
            else:
                ebase = sid * ept + blk * eblk
            pltpu.sync_copy(src_hbm.at[pl.ds(ebase, eblk)], sidx)
            pltpu.sync_copy(dst_hbm.at[pl.ds(ebase, eblk)], didx)

            gather(0, rows_a, sem_a)

            def pair(p, _):
                i = p * 2
                gather(i + 1, rows_b, sem_b)
                wait_gather(i, rows_a, sem_a)
                scatter(i, rows_a)

                @pl.when(i + 2 < nchunk)
                def _next():
                    gather(i + 2, rows_a, sem_a)
                wait_gather(i + 1, rows_b, sem_b)
                scatter(i + 1, rows_b)
                return 0
            lax.fori_loop(0, nchunk // 2, pair, 0)
            wait_gather(nchunk - 1, rows_a, sem_a)
            scatter(nchunk - 1, rows_a)
        plsc.subcore_barrier()

        cbase = sid * 624
        pltpu.sync_copy(accum.at[pl.ds(cbase, 624)],
                        out_hbm.at[pl.ds(off0 + cbase, 624)])

        @pl.when(sid == _NS - 1)
        def _tail():
            pltpu.sync_copy(accum.at[pl.ds(9984, 16)],
                            out_hbm.at[pl.ds(off0 + 9984, 16)])

    return agg


def kernel(features, edge_index, W0, W1, W2):
    src = edge_index[0].astype(jnp.int32)
    dst = edge_index[1].astype(jnp.int32)
    fagg = _make_agg(False)
    eagg = _make_agg(True)
    hp = _mm_first(features, W0)
    a0 = fagg(hp[0], hp[1], src, dst).reshape(_NC, _N, 128)
    hp = _mm_mid(a0, W1)
    a1 = fagg(hp[0], hp[1], src, dst).reshape(_NC, _N, 128)
    h2 = _mm_full(a1, W2)
    a2 = eagg(h2, h2, src, dst).reshape(_NC, _N, 128)
    return _final_tanh_sum(a2)


# speedup vs baseline: 1.0527x; 1.0067x over previous
import functools

import jax
import jax.numpy as jnp
from jax import lax
from jax.experimental import pallas as pl
from jax.experimental.pallas import tpu as pltpu
from jax.experimental.pallas import tpu_sc as plsc

_N = 10000
_E = 320000
_NC = 2
_NS = 16
_RPT = _N // _NS
_R = 2000


def _mm_first_body(x_ref, w_ref, o_ref):
    h = jnp.dot(x_ref[...], w_ref[...], preferred_element_type=jnp.float32)
    o_ref[0, :, :] = h[:, :128]
    o_ref[1, :, :] = h[:, 128:]


def _mm_first(x, w):
    return pl.pallas_call(
        _mm_first_body,
        grid=(_N // _R,),
        in_specs=[pl.BlockSpec((_R, 128), lambda i: (i, 0)),
                  pl.BlockSpec((128, 256), lambda i: (0, 0))],
        out_specs=pl.BlockSpec((2, _R, 128), lambda i: (0, i, 0)),
        out_shape=jax.ShapeDtypeStruct((2, _N, 128), jnp.float32),
    )(x, w)


def _mm_mid_body(a_ref, w_ref, o_ref):
    x = jnp.concatenate([a_ref[0, :, :], a_ref[1, :, :]], axis=1)
    h = jnp.dot(jnp.tanh(x), w_ref[...], preferred_element_type=jnp.float32)
    o_ref[0, :, :] = h[:, :128]
    o_ref[1, :, :] = h[:, 128:]


def _mm_mid(a, w):
    return pl.pallas_call(
        _mm_mid_body,
        grid=(_N // _R,),
        in_specs=[pl.BlockSpec((2, _R, 128), lambda i: (0, i, 0)),
                  pl.BlockSpec((256, 256), lambda i: (0, 0))],
        out_specs=pl.BlockSpec((2, _R, 128), lambda i: (0, i, 0)),
        out_shape=jax.ShapeDtypeStruct((2, _N, 128), jnp.float32),
    )(a, w)


def _mm_full_body(a_ref, w_ref, o_ref):
    x = jnp.concatenate([a_ref[0, :, :], a_ref[1, :, :]], axis=1)
    o_ref[...] = jnp.dot(jnp.tanh(x), w_ref[...],
                         preferred_element_type=jnp.float32)


def _mm_full(a, w):
    dout = w.shape[1]
    return pl.pallas_call(
        _mm_full_body,
        grid=(_N // _R,),
        in_specs=[pl.BlockSpec((2, _R, 128), lambda i: (0, i, 0)),
                  pl.BlockSpec(w.shape, lambda i: (0, 0))],
        out_specs=pl.BlockSpec((_R, dout), lambda i: (i, 0)),
        out_shape=jax.ShapeDtypeStruct((_N, dout), jnp.float32),
    )(a, w)


def _tanh_sum_body(a_ref, o_ref):
    o_ref[...] = jnp.tanh(a_ref[0, :, :] + a_ref[1, :, :])


def _final_tanh_sum(a):
    return pl.pallas_call(
        _tanh_sum_body,
        grid=(_N // _R,),
        in_specs=[pl.BlockSpec((2, _R, 128), lambda i: (0, i, 0))],
        out_specs=pl.BlockSpec((_R, 128), lambda i: (i, 0)),
        out_shape=jax.ShapeDtypeStruct((_N, 128), jnp.float32),
    )(a)


@functools.lru_cache(maxsize=None)
def _make_agg(edge_split):
    chunk = 80 if edge_split else 160
    nchunk = 25
    eblk = nchunk * chunk
    ept = (_E // (_NC * _NS)) if edge_split else (_E // _NS)
    nblk = ept // eblk
    mesh = plsc.VectorSubcoreMesh(core_axis_name="c", subcore_axis_name="s")

    @functools.partial(
        pl.kernel, mesh=mesh,
        out_type=jax.ShapeDtypeStruct((_NC * _N, 128), jnp.float32),
        scratch_types=[
            pltpu.VMEM((eblk,), jnp.int32),
            pltpu.VMEM((eblk,), jnp.int32),
            pltpu.VMEM((chunk, 128), jnp.float32),
            pltpu.VMEM((chunk, 128), jnp.float32),
            pltpu.VMEM_SHARED((_N, 128), jnp.float32),
            pltpu.SemaphoreType.DMA,
            pltpu.SemaphoreType.DMA,
        ],
    )
    def agg(h0_hbm, h1_hbm, src_hbm, dst_hbm, out_hbm, sidx, didx, rows_a,
            rows_b, accum, sem_a, sem_b):
        cid = lax.axis_index("c")
        sid = lax.axis_index("s")
        off0 = cid * _N

        def gather(i, buf, sem):
            idx = sidx.at[pl.ds(i * chunk, chunk)]

            @pl.when(cid == 0)
            def _g0():
                pltpu.make_async_copy(h0_hbm.at[idx], buf, sem).start()

            @pl.when(cid == 1)
            def _g1():
                pltpu.make_async_copy(h1_hbm.at[idx], buf, sem).start()

        def wait_gather(i, buf, sem):
            idx = sidx.at[pl.ds(i * chunk, chunk)]
            pltpu.make_async_copy(h0_hbm.at[idx], buf, sem).wait()

        def scatter(i, buf):
            idx = didx.at[pl.ds(i * chunk, chunk)]
            pltpu.sync_copy(buf, accum.at[idx], add=True)

        def ebase(blk):
            if edge_split:
                return cid * (_E // _NC) + sid * ept + blk * eblk
            return sid * ept + blk * eblk

        def run_block(blk, b0, s0, b1, s1, preloaded):
            if not preloaded:
                pltpu.sync_copy(src_hbm.at[pl.ds(ebase(blk), eblk)], sidx)
                pltpu.sync_copy(dst_hbm.at[pl.ds(ebase(blk), eblk)], didx)
                gather(0, b0, s0)

            def pair(p, _):
                i = p * 2
                gather(i + 1, b1, s1)
                wait_gather(i, b0, s0)
                scatter(i, b0)

                @pl.when(i + 2 < nchunk)
                def _next():
                    gather(i + 2, b0, s0)
                wait_gather(i + 1, b1, s1)
                scatter(i + 1, b1)
                return 0
            lax.fori_loop(0, nchunk // 2, pair, 0)
            wait_gather(nchunk - 1, b0, s0)
            scatter(nchunk - 1, b0)

        pltpu.sync_copy(src_hbm.at[pl.ds(ebase(0), eblk)], sidx)
        pltpu.sync_copy(dst_hbm.at[pl.ds(ebase(0), eblk)], didx)
        gather(0, rows_b, sem_b)

        def zrow(r, _):
            def zcol(j, _):
                rows_a[r, pl.ds(j * 16, 16)] = jnp.zeros((16,), jnp.float32)
                return 0
            return lax.fori_loop(0, 128 // 16, zcol, 0)
        lax.fori_loop(0, chunk, zrow, 0)
        rbase = sid * _RPT
        nz = _RPT // chunk
        for z in range(nz):
            pltpu.sync_copy(rows_a, accum.at[pl.ds(rbase + z * chunk, chunk)])
        rem = _RPT - nz * chunk
        pltpu.sync_copy(rows_a.at[pl.ds(0, rem)],
                        accum.at[pl.ds(rbase + nz * chunk, rem)])
        plsc.subcore_barrier()

        run_block(0, rows_b, sem_b, rows_a, sem_a, True)
        for blk in range(1, nblk):
            run_block(blk, rows_a, sem_a, rows_b, sem_b, False)
        plsc.subcore_barrier()

        cbase = sid * 624
        pltpu.sync_copy(accum.at[pl.ds(cbase, 624)],
                        out_hbm.at[pl.ds(off0 + cbase, 624)])

        @pl.when(sid == _NS - 1)
        def _tail():
            pltpu.sync_copy(accum.at[pl.ds(9984, 16)],
                            out_hbm.at[pl.ds(off0 + 9984, 16)])

    return agg


def kernel(features, edge_index, W0, W1, W2):
    src = edge_index[0].astype(jnp.int32)
    dst = edge_index[1].astype(jnp.int32)
    fagg = _make_agg(False)
    eagg = _make_agg(True)
    hp = _mm_first(features, W0)
    a0 = fagg(hp[0], hp[1], src, dst).reshape(_NC, _N, 128)
    hp = _mm_mid(a0, W1)
    a1 = fagg(hp[0], hp[1], src, dst).reshape(_NC, _N, 128)
    h2 = _mm_full(a1, W2)
    a2 = eagg(h2, h2, src, dst).reshape(_NC, _N, 128)
    return _final_tanh_sum(a2)
